# baseline (speedup 1.0000x reference)
import functools

import jax
import jax.numpy as jnp
from jax import lax
from jax.experimental import pallas as pl
from jax.experimental.pallas import tpu as pltpu
from jax.experimental.pallas import tpu_sc as plsc

_LANES = 16


@functools.lru_cache(maxsize=None)
def _make_sc_add(B: int, S: int, D: int):
    info = plsc.get_sparse_core_info()
    NC, NS = info.num_cores, info.num_subcores
    NW = NC * NS

    rows_per_w = S // NW
    T_ROWS = 32
    n_tiles = rows_per_w // T_ROWS
    TILE = T_ROWS * D
    assert S % NW == 0 and rows_per_w % T_ROWS == 0 and D % _LANES == 0

    mesh = plsc.VectorSubcoreMesh(core_axis_name="c", subcore_axis_name="s")

    @functools.partial(
        pl.kernel,
        mesh=mesh,
        out_type=jax.ShapeDtypeStruct((B * S * D,), jnp.float32),
        scratch_types=[
            pltpu.VMEM((TILE,), jnp.float32),
            pltpu.VMEM((TILE,), jnp.float32),
        ],
    )
    def k(x_hbm, pe_hbm, out_hbm, pebuf, xbuf):
        wid = lax.axis_index("s") * NC + lax.axis_index("c")
        w_base = wid * rows_per_w * D
        for t in range(n_tiles):
            pe_off = w_base + t * TILE
            pltpu.sync_copy(pe_hbm.at[pl.ds(pe_off, TILE)], pebuf)
            for b in range(B):
                x_off = b * S * D + pe_off
                pltpu.sync_copy(x_hbm.at[pl.ds(x_off, TILE)], xbuf)

                def add_body(i, _, unroll=8):
                    for u in range(unroll):
                        sl = pl.ds((i * unroll + u) * _LANES, _LANES)
                        xbuf[sl] = xbuf[sl] + pebuf[sl]
                    return _

                lax.fori_loop(0, TILE // (_LANES * 8), add_body, 0)
                pltpu.sync_copy(xbuf, out_hbm.at[pl.ds(x_off, TILE)])

    return k


def kernel(x, pe_emb):
    B, S, D = x.shape
    k = _make_sc_add(B, S, D)
    out = k(x.reshape(-1), pe_emb.reshape(-1))
    return out.reshape(B, S, D)


# speedup vs baseline: 1.1421x; 1.1421x over previous
import functools

import jax
import jax.numpy as jnp
from jax import lax
from jax.experimental import pallas as pl
from jax.experimental.pallas import tpu as pltpu
from jax.experimental.pallas import tpu_sc as plsc

_LANES = 16


@functools.lru_cache(maxsize=None)
def _make_sc_add(B: int, S: int, D: int):
    info = plsc.get_sparse_core_info()
    NC, NS = info.num_cores, info.num_subcores
    NW = NC * NS

    rows_per_w = S // NW
    T_ROWS = 32
    n_tiles = rows_per_w // T_ROWS
    TILE = T_ROWS * D
    assert S % NW == 0 and rows_per_w % T_ROWS == 0 and D % _LANES == 0

    mesh = plsc.VectorSubcoreMesh(core_axis_name="c", subcore_axis_name="s")

    @functools.partial(
        pl.kernel,
        mesh=mesh,
        out_type=jax.ShapeDtypeStruct((B * S * D,), jnp.float32),
        scratch_types=[
            pltpu.VMEM((TILE,), jnp.float32),
            pltpu.VMEM((TILE,), jnp.float32),
            pltpu.VMEM((TILE,), jnp.float32),
            pltpu.SemaphoreType.DMA,
            pltpu.SemaphoreType.DMA,
            pltpu.SemaphoreType.DMA,
            pltpu.SemaphoreType.DMA,
            pltpu.SemaphoreType.DMA,
        ],
    )
    def k(x_hbm, pe_hbm, out_hbm, pebuf, xb0, xb1, ls0, ls1, ss0, ss1, pes):
        xb = (xb0, xb1)
        ls = (ls0, ls1)
        ss = (ss0, ss1)
        wid = lax.axis_index("s") * NC + lax.axis_index("c")
        w_base = wid * rows_per_w * D

        steps = [(t, b) for t in range(n_tiles) for b in range(B)]
        n = len(steps)

        def x_off(i):
            t, b = steps[i]
            return b * S * D + w_base + t * TILE

        def start_load(i):
            p = i % 2
            return pltpu.async_copy(
                x_hbm.at[pl.ds(x_off(i), TILE)], xb[p], ls[p])

        h_store = [None] * n
        h_pe = pltpu.async_copy(pe_hbm.at[pl.ds(w_base, TILE)], pebuf, pes)
        h_load = start_load(0)
        for i in range(n):
            t, b = steps[i]
            p = i % 2
            if i + 1 < n:
                if i - 1 >= 0:
                    h_store[i - 1].wait()
                nxt = start_load(i + 1)
            h_load.wait()
            if i + 1 < n:
                h_load = nxt
            if b == 0:
                h_pe.wait()
            xbp = xb[p]

            @plsc.parallel_loop(0, TILE // _LANES, unroll=8)
            def add_body(j):
                sl = pl.ds(j * _LANES, _LANES)
                xbp[sl] = xbp[sl] + pebuf[sl]

            h_store[i] = pltpu.async_copy(
                xbp, out_hbm.at[pl.ds(x_off(i), TILE)], ss[p])
            if b == B - 1 and t + 1 < n_tiles:
                h_pe = pltpu.async_copy(
                    pe_hbm.at[pl.ds(w_base + (t + 1) * TILE, TILE)],
                    pebuf, pes)
        h_store[n - 2].wait()
        h_store[n - 1].wait()

    return k


def kernel(x, pe_emb):
    B, S, D = x.shape
    k = _make_sc_add(B, S, D)
    out = k(x.reshape(-1), pe_emb.reshape(-1))
    return out.reshape(B, S, D)


# speedup vs baseline: 2.6367x; 2.3087x over previous
import functools

import jax
import jax.numpy as jnp
from jax import lax
from jax.experimental import pallas as pl
from jax.experimental.pallas import tpu as pltpu
from jax.experimental.pallas import tpu_sc as plsc

_LANES = 16


@functools.lru_cache(maxsize=None)
def _make_sc_add(B: int, S: int, D: int):
    info = plsc.get_sparse_core_info()
    NC, NS = info.num_cores, info.num_subcores
    NW = NC * NS

    rows_per_w = S // NW
    T_ROWS = 32
    n_tiles = rows_per_w // T_ROWS
    TILE = T_ROWS * D
    assert S % NW == 0 and rows_per_w % T_ROWS == 0 and D % _LANES == 0

    mesh = plsc.VectorSubcoreMesh(core_axis_name="c", subcore_axis_name="s")

    @functools.partial(
        pl.kernel,
        mesh=mesh,
        out_type=jax.ShapeDtypeStruct((B * S, D), jnp.float32),
        scratch_types=[
            pltpu.VMEM((T_ROWS, D), jnp.float32),
            pltpu.VMEM((T_ROWS, D), jnp.float32),
            pltpu.VMEM((T_ROWS, D), jnp.float32),
            pltpu.SemaphoreType.DMA,
            pltpu.SemaphoreType.DMA,
            pltpu.SemaphoreType.DMA,
            pltpu.SemaphoreType.DMA,
            pltpu.SemaphoreType.DMA,
        ],
    )
    def k(x_hbm, pe_hbm, out_hbm, pebuf, xb0, xb1, ls0, ls1, ss0, ss1, pes):
        xb = (xb0, xb1)
        ls = (ls0, ls1)
        ss = (ss0, ss1)
        wid = lax.axis_index("s") * NC + lax.axis_index("c")
        w_row = wid * rows_per_w

        steps = [(t, b) for t in range(n_tiles) for b in range(B)]
        n = len(steps)

        def x_row(i):
            t, b = steps[i]
            return b * S + w_row + t * T_ROWS

        def start_load(i):
            p = i % 2
            return pltpu.async_copy(
                x_hbm.at[pl.ds(x_row(i), T_ROWS)], xb[p], ls[p])

        h_store = [None] * n
        h_pe = pltpu.async_copy(pe_hbm.at[pl.ds(w_row, T_ROWS)], pebuf, pes)
        h_load = start_load(0)
        for i in range(n):
            t, b = steps[i]
            p = i % 2
            if i + 1 < n:
                if i - 1 >= 0:
                    h_store[i - 1].wait()
                nxt = start_load(i + 1)
            h_load.wait()
            if i + 1 < n:
                h_load = nxt
            if b == 0:
                h_pe.wait()
            xbp = xb[p]

            @plsc.parallel_loop(0, T_ROWS, unroll=1)
            def add_body(r):
                for j in range(D // _LANES):
                    sl = pl.ds(j * _LANES, _LANES)
                    xbp[r, sl] = xbp[r, sl] + pebuf[r, sl]

            h_store[i] = pltpu.async_copy(
                xbp, out_hbm.at[pl.ds(x_row(i), T_ROWS)], ss[p])
            if b == B - 1 and t + 1 < n_tiles:
                h_pe = pltpu.async_copy(
                    pe_hbm.at[pl.ds(w_row + (t + 1) * T_ROWS, T_ROWS)],
                    pebuf, pes)
        h_store[n - 2].wait()
        h_store[n - 1].wait()

    return k


def kernel(x, pe_emb):
    B, S, D = x.shape
    k = _make_sc_add(B, S, D)
    out = k(x.reshape(B * S, D), pe_emb)
    return out.reshape(B, S, D)
